# baseline (speedup 1.0000x reference)
import jax
import jax.numpy as jnp
from jax import lax
from jax.experimental import pallas as pl
from jax.experimental.pallas import tpu as pltpu
from jax.experimental.pallas import tpu_sc as plsc

E = 8
D = 1024
F = 512
T = 2048
TB = 256
TT = 256
MAXT = 15
NPAD = MAXT * TT
NW = 32
TPW = T // NW


def _router_body(x_ref, wr_ref, val_ref, tile_ref, nt_ref, dest_ref,
                 counts_sc, e_sc, rank_sc):
    b = pl.program_id(0)

    @pl.when(b == 0)
    def _():
        counts_sc[...] = jnp.zeros((1, E), jnp.float32)

    xb = x_ref[...]
    logits = jnp.dot(xb, wr_ref[...], preferred_element_type=jnp.float32)
    m = jnp.max(logits, axis=1, keepdims=True)
    p = jnp.exp(logits - m)
    probs = p / jnp.sum(p, axis=1, keepdims=True)
    pmax = jnp.max(probs, axis=1, keepdims=True)
    iota_e = lax.broadcasted_iota(jnp.int32, (TB, E), 1)
    idx = jnp.min(jnp.where(probs >= pmax, iota_e, E), axis=1, keepdims=True)
    onehot = (iota_e == idx).astype(jnp.float32)

    r_i = lax.broadcasted_iota(jnp.int32, (TB, TB), 0)
    c_i = lax.broadcasted_iota(jnp.int32, (TB, TB), 1)
    ltri = (c_i < r_i).astype(jnp.float32)
    rank_blk = jnp.dot(ltri, onehot, preferred_element_type=jnp.float32)
    running = counts_sc[...]
    rank_glb = jnp.sum((rank_blk + running) * onehot, axis=1, keepdims=True)
    counts_sc[...] = running + jnp.sum(onehot, axis=0, keepdims=True)

    val_ref[...] = pmax
    e_sc[pl.ds(b * TB, TB), :] = idx
    rank_sc[pl.ds(b * TB, TB), :] = rank_glb.astype(jnp.int32)

    @pl.when(b == T // TB - 1)
    def _():
        counts_i = counts_sc[...].astype(jnp.int32)
        pc = ((counts_i + (TT - 1)) // TT) * TT
        a_i = lax.broadcasted_iota(jnp.int32, (E, E), 0)
        b_i = lax.broadcasted_iota(jnp.int32, (E, E), 1)
        excl = (a_i < b_i).astype(jnp.float32)
        starts = jnp.dot(pc.astype(jnp.float32), excl,
                         preferred_element_type=jnp.float32).astype(jnp.int32)
        ends = starts + pc
        ts = lax.broadcasted_iota(jnp.int32, (1, 16), 1) * TT
        te = jnp.zeros((1, 16), jnp.int32)
        for e in range(E):
            te = te + (ts >= ends[:, e:e + 1]).astype(jnp.int32)
        tile_ref[...] = jnp.minimum(te, E - 1)
        nt_ref[...] = jnp.sum(pc, axis=1, keepdims=True) // TT
        iota_all = lax.broadcasted_iota(jnp.int32, (T, E), 1)
        oh_all = (iota_all == e_sc[...]).astype(jnp.int32)
        dest_ref[...] = rank_sc[...] + jnp.sum(
            oh_all * starts, axis=1, keepdims=True)


def _run_router(x2, W_router):
    return pl.pallas_call(
        _router_body,
        grid=(T // TB,),
        in_specs=[
            pl.BlockSpec((TB, D), lambda b: (b, 0)),
            pl.BlockSpec((D, E), lambda b: (0, 0)),
        ],
        out_specs=[
            pl.BlockSpec((TB, 1), lambda b: (b, 0)),
            pl.BlockSpec((1, 16), lambda b: (0, 0)),
            pl.BlockSpec((1, 1), lambda b: (0, 0)),
            pl.BlockSpec((T, 1), lambda b: (0, 0)),
        ],
        out_shape=[
            jax.ShapeDtypeStruct((T, 1), jnp.float32),
            jax.ShapeDtypeStruct((1, 16), jnp.int32),
            jax.ShapeDtypeStruct((1, 1), jnp.int32),
            jax.ShapeDtypeStruct((T, 1), jnp.int32),
        ],
        scratch_shapes=[
            pltpu.VMEM((1, E), jnp.float32),
            pltpu.VMEM((T, 1), jnp.int32),
            pltpu.VMEM((T, 1), jnp.int32),
        ],
        compiler_params=pltpu.CompilerParams(
            dimension_semantics=("arbitrary",)),
    )(x2, W_router)


def _scatter_body(x_hbm, dest_hbm, sorted_hbm, dest_v, rows_v, sem):
    wid = lax.axis_index("s") * 2 + lax.axis_index("c")
    base = wid * TPW
    pltpu.sync_copy(dest_hbm.at[pl.ds(base, TPW)], dest_v)
    pltpu.sync_copy(x_hbm.at[pl.ds(base, TPW)], rows_v)
    pltpu.async_copy(rows_v, sorted_hbm.at[dest_v], sem).wait()


def _sc_scatter(x2, dest1):
    mesh = plsc.VectorSubcoreMesh(core_axis_name="c", subcore_axis_name="s")
    fn = pl.kernel(
        _scatter_body,
        out_type=jax.ShapeDtypeStruct((NPAD, D), jnp.float32),
        mesh=mesh,
        scratch_types=[
            pltpu.VMEM((TPW,), jnp.int32),
            pltpu.VMEM((TPW, D), jnp.float32),
            pltpu.SemaphoreType.DMA,
        ],
    )
    return fn(x2, dest1)


def _expert_body(te_ref, nt_ref, xs_ref, w1_ref, w2_ref, xb_ref, wgu_ref,
                 wd_ref, wsg_ref, o_ref, sh_ref):
    t = pl.program_id(0)

    @pl.when(t < nt_ref[0])
    def _():
        xb = xs_ref[...]
        h = jnp.dot(xb, w1_ref[0], preferred_element_type=jnp.float32)
        h = jax.nn.silu(h)
        o_ref[...] = jnp.dot(h, w2_ref[0], preferred_element_type=jnp.float32)

    @pl.when(t < T // TB)
    def _():
        xb = xb_ref[...]
        gu = jnp.dot(xb, wgu_ref[...], preferred_element_type=jnp.float32)
        g = gu[:, :F]
        u = gu[:, F:]
        sh = jnp.dot(jax.nn.silu(g) * u, wd_ref[...],
                     preferred_element_type=jnp.float32)
        sg = jax.nn.sigmoid(jnp.dot(xb, wsg_ref[...],
                                    preferred_element_type=jnp.float32))
        sh_ref[...] = sg * sh


def _run_experts(tile_e16, ntiles1, sorted_x, W1b, W2b,
                 x2, W_gate_up, W_down, W_shared_gate):
    nb = T // TB
    grid_spec = pltpu.PrefetchScalarGridSpec(
        num_scalar_prefetch=2,
        grid=(MAXT,),
        in_specs=[
            pl.BlockSpec((TT, D),
                         lambda t, te, nt: (jnp.minimum(t, nt[0] - 1), 0)),
            pl.BlockSpec((1, D, F), lambda t, te, nt: (te[t], 0, 0)),
            pl.BlockSpec((1, F, D), lambda t, te, nt: (te[t], 0, 0)),
            pl.BlockSpec((TB, D),
                         lambda t, te, nt: (jnp.minimum(t, nb - 1), 0)),
            pl.BlockSpec((D, 2 * F), lambda t, te, nt: (0, 0)),
            pl.BlockSpec((F, D), lambda t, te, nt: (0, 0)),
            pl.BlockSpec((D, 1), lambda t, te, nt: (0, 0)),
        ],
        out_specs=[
            pl.BlockSpec((TT, D),
                         lambda t, te, nt: (jnp.minimum(t, nt[0] - 1), 0)),
            pl.BlockSpec((TB, D),
                         lambda t, te, nt: (jnp.minimum(t, nb - 1), 0)),
        ],
    )
    return pl.pallas_call(
        _expert_body,
        grid_spec=grid_spec,
        out_shape=[
            jax.ShapeDtypeStruct((NPAD, D), jnp.float32),
            jax.ShapeDtypeStruct((T, D), jnp.float32),
        ],
        compiler_params=pltpu.CompilerParams(
            dimension_semantics=("arbitrary",)),
    )(tile_e16, ntiles1, sorted_x, W1b, W2b,
      x2, W_gate_up, W_down, W_shared_gate)


def _finish_body(routed_hbm, dest_hbm, shared_hbm, val_hbm, out_hbm,
                 dest_v, val_v, rows_v, sh_v, sem):
    wid = lax.axis_index("s") * 2 + lax.axis_index("c")
    base = wid * TPW
    ch_rows = TPW // 2
    pltpu.sync_copy(val_hbm.at[pl.ds(base, TPW)], val_v)
    for ch in range(2):
        cb = base + ch * ch_rows
        pltpu.sync_copy(dest_hbm.at[pl.ds(cb, ch_rows)], dest_v.at[ch])
        pltpu.async_copy(routed_hbm.at[dest_v.at[ch]], rows_v, sem).wait()
        pltpu.sync_copy(shared_hbm.at[pl.ds(cb, ch_rows)], sh_v)

        def row_body(r, carry, _ch=ch):
            off = pl.multiple_of((_ch * ch_rows) + (r // 16) * 16, 16)
            vchunk = val_v[pl.ds(off, 16)]
            bv = lax.gather(
                vchunk, jnp.full((16, 1), r % 16, jnp.int32),
                dimension_numbers=lax.GatherDimensionNumbers(
                    offset_dims=(), collapsed_slice_dims=(0,),
                    start_index_map=(0,)),
                slice_sizes=(1,),
                mode=lax.GatherScatterMode.PROMISE_IN_BOUNDS)
            for c in range(D // 16):
                sl = pl.ds(c * 16, 16)
                rows_v[r, sl] = bv * rows_v[r, sl] + sh_v[r, sl]
            return carry

        lax.fori_loop(0, ch_rows, row_body, 0)
        pltpu.sync_copy(rows_v, out_hbm.at[pl.ds(cb, ch_rows)])


def _sc_finish(routed_sorted, dest, shared, val1):
    mesh = plsc.VectorSubcoreMesh(core_axis_name="c", subcore_axis_name="s")
    fn = pl.kernel(
        _finish_body,
        out_type=jax.ShapeDtypeStruct((T, D), jnp.float32),
        mesh=mesh,
        scratch_types=[
            pltpu.VMEM((2, TPW // 2), jnp.int32),
            pltpu.VMEM((TPW,), jnp.float32),
            pltpu.VMEM((TPW // 2, D), jnp.float32),
            pltpu.VMEM((TPW // 2, D), jnp.float32),
            pltpu.SemaphoreType.DMA,
        ],
    )
    return fn(routed_sorted, dest, shared, val1)


def kernel(x, W_router, W1, W2, W_gate_up, W_down, W_shared_gate):
    Bb, Tt, Dd = x.shape
    x2 = x.reshape(Tt, Dd)
    val2, tile_e, ntl, dest2 = _run_router(x2, W_router)
    dest = dest2.reshape(Tt)
    sorted_x = _sc_scatter(x2, dest)
    routed_sorted, shared = _run_experts(
        tile_e.reshape(16), ntl.reshape(1), sorted_x, W1, W2,
        x2, W_gate_up, W_down, W_shared_gate)
    out = _sc_finish(routed_sorted, dest, shared, val2.reshape(Tt))
    return out.reshape(Bb, Tt, Dd)


# speedup vs baseline: 1.0376x; 1.0376x over previous
import jax
import jax.numpy as jnp
from jax import lax
from jax.experimental import pallas as pl
from jax.experimental.pallas import tpu as pltpu
from jax.experimental.pallas import tpu_sc as plsc

E = 8
D = 1024
F = 512
T = 2048
TB = 256
TT = 256
MAXT = 15
NPAD = MAXT * TT
NW = 32
TPW = T // NW


def _router_body(x_ref, wr_ref, val_ref, tile_ref, nt_ref, dest_ref,
                 counts_sc, e_sc, rank_sc):
    b = pl.program_id(0)

    @pl.when(b == 0)
    def _():
        counts_sc[...] = jnp.zeros((1, E), jnp.float32)

    xb = x_ref[...]
    logits = jnp.dot(xb, wr_ref[...], preferred_element_type=jnp.float32)
    m = jnp.max(logits, axis=1, keepdims=True)
    p = jnp.exp(logits - m)
    probs = p / jnp.sum(p, axis=1, keepdims=True)
    pmax = jnp.max(probs, axis=1, keepdims=True)
    iota_e = lax.broadcasted_iota(jnp.int32, (TB, E), 1)
    idx = jnp.min(jnp.where(probs >= pmax, iota_e, E), axis=1, keepdims=True)
    onehot = (iota_e == idx).astype(jnp.float32)

    r_i = lax.broadcasted_iota(jnp.int32, (TB, TB), 0)
    c_i = lax.broadcasted_iota(jnp.int32, (TB, TB), 1)
    ltri = (c_i < r_i).astype(jnp.float32)
    rank_blk = jnp.dot(ltri, onehot, preferred_element_type=jnp.float32)
    running = counts_sc[...]
    rank_glb = jnp.sum((rank_blk + running) * onehot, axis=1, keepdims=True)
    counts_sc[...] = running + jnp.sum(onehot, axis=0, keepdims=True)

    val_ref[...] = pmax
    e_sc[pl.ds(b * TB, TB), :] = idx
    rank_sc[pl.ds(b * TB, TB), :] = rank_glb.astype(jnp.int32)

    @pl.when(b == T // TB - 1)
    def _():
        counts_i = counts_sc[...].astype(jnp.int32)
        pc = ((counts_i + (TT - 1)) // TT) * TT
        a_i = lax.broadcasted_iota(jnp.int32, (E, E), 0)
        b_i = lax.broadcasted_iota(jnp.int32, (E, E), 1)
        excl = (a_i < b_i).astype(jnp.float32)
        starts = jnp.dot(pc.astype(jnp.float32), excl,
                         preferred_element_type=jnp.float32).astype(jnp.int32)
        ends = starts + pc
        ts = lax.broadcasted_iota(jnp.int32, (1, 16), 1) * TT
        te = jnp.zeros((1, 16), jnp.int32)
        for e in range(E):
            te = te + (ts >= ends[:, e:e + 1]).astype(jnp.int32)
        tile_ref[...] = jnp.minimum(te, E - 1)
        nt_ref[...] = jnp.sum(pc, axis=1, keepdims=True) // TT
        iota_all = lax.broadcasted_iota(jnp.int32, (T, E), 1)
        oh_all = (iota_all == e_sc[...]).astype(jnp.int32)
        dest_ref[...] = rank_sc[...] + jnp.sum(
            oh_all * starts, axis=1, keepdims=True)


def _run_router(x2, W_router):
    return pl.pallas_call(
        _router_body,
        grid=(T // TB,),
        in_specs=[
            pl.BlockSpec((TB, D), lambda b: (b, 0)),
            pl.BlockSpec((D, E), lambda b: (0, 0)),
        ],
        out_specs=[
            pl.BlockSpec((TB, 1), lambda b: (b, 0)),
            pl.BlockSpec((1, 16), lambda b: (0, 0)),
            pl.BlockSpec((1, 1), lambda b: (0, 0)),
            pl.BlockSpec((T, 1), lambda b: (0, 0)),
        ],
        out_shape=[
            jax.ShapeDtypeStruct((T, 1), jnp.float32),
            jax.ShapeDtypeStruct((1, 16), jnp.int32),
            jax.ShapeDtypeStruct((1, 1), jnp.int32),
            jax.ShapeDtypeStruct((T, 1), jnp.int32),
        ],
        scratch_shapes=[
            pltpu.VMEM((1, E), jnp.float32),
            pltpu.VMEM((T, 1), jnp.int32),
            pltpu.VMEM((T, 1), jnp.int32),
        ],
        compiler_params=pltpu.CompilerParams(
            dimension_semantics=("arbitrary",)),
    )(x2, W_router)


def _scatter_body(x_hbm, dest_hbm, sorted_hbm, dest_v, rows_v, sem):
    wid = lax.axis_index("s") * 2 + lax.axis_index("c")
    base = wid * TPW
    pltpu.sync_copy(dest_hbm.at[pl.ds(base, TPW)], dest_v)
    pltpu.sync_copy(x_hbm.at[pl.ds(base, TPW)], rows_v)
    pltpu.async_copy(rows_v, sorted_hbm.at[dest_v], sem).wait()


def _sc_scatter(x2, dest1):
    mesh = plsc.VectorSubcoreMesh(core_axis_name="c", subcore_axis_name="s")
    fn = pl.kernel(
        _scatter_body,
        out_type=jax.ShapeDtypeStruct((NPAD, D), jnp.float32),
        mesh=mesh,
        scratch_types=[
            pltpu.VMEM((TPW,), jnp.int32),
            pltpu.VMEM((TPW, D), jnp.float32),
            pltpu.SemaphoreType.DMA,
        ],
    )
    return fn(x2, dest1)


def _expert_body(te_ref, nt_ref, xs_ref, w1_ref, w2_ref, xb_ref, wgu_ref,
                 wd_ref, wsg_ref, o_ref, sh_ref):
    t = pl.program_id(0)

    @pl.when(t < nt_ref[0])
    def _():
        xb = xs_ref[...]
        h = jnp.dot(xb, w1_ref[0], preferred_element_type=jnp.float32)
        h = jax.nn.silu(h)
        o_ref[...] = jnp.dot(h, w2_ref[0], preferred_element_type=jnp.float32)

    @pl.when(t < T // TB)
    def _():
        xb = xb_ref[...]
        gu = jnp.dot(xb, wgu_ref[...], preferred_element_type=jnp.float32)
        g = gu[:, :F]
        u = gu[:, F:]
        sh = jnp.dot(jax.nn.silu(g) * u, wd_ref[...],
                     preferred_element_type=jnp.float32)
        sg = jax.nn.sigmoid(jnp.dot(xb, wsg_ref[...],
                                    preferred_element_type=jnp.float32))
        sh_ref[...] = sg * sh


def _run_experts(tile_e16, ntiles1, sorted_x, W1b, W2b,
                 x2, W_gate_up, W_down, W_shared_gate):
    nb = T // TB
    grid_spec = pltpu.PrefetchScalarGridSpec(
        num_scalar_prefetch=2,
        grid=(MAXT,),
        in_specs=[
            pl.BlockSpec((TT, D),
                         lambda t, te, nt: (jnp.minimum(t, nt[0] - 1), 0)),
            pl.BlockSpec((1, D, F), lambda t, te, nt: (te[t], 0, 0)),
            pl.BlockSpec((1, F, D), lambda t, te, nt: (te[t], 0, 0)),
            pl.BlockSpec((TB, D),
                         lambda t, te, nt: (jnp.minimum(t, nb - 1), 0)),
            pl.BlockSpec((D, 2 * F), lambda t, te, nt: (0, 0)),
            pl.BlockSpec((F, D), lambda t, te, nt: (0, 0)),
            pl.BlockSpec((D, 1), lambda t, te, nt: (0, 0)),
        ],
        out_specs=[
            pl.BlockSpec((TT, D),
                         lambda t, te, nt: (jnp.minimum(t, nt[0] - 1), 0)),
            pl.BlockSpec((TB, D),
                         lambda t, te, nt: (jnp.minimum(t, nb - 1), 0)),
        ],
    )
    return pl.pallas_call(
        _expert_body,
        grid_spec=grid_spec,
        out_shape=[
            jax.ShapeDtypeStruct((NPAD, D), jnp.float32),
            jax.ShapeDtypeStruct((T, D), jnp.float32),
        ],
        compiler_params=pltpu.CompilerParams(
            dimension_semantics=("arbitrary",)),
    )(tile_e16, ntiles1, sorted_x, W1b, W2b,
      x2, W_gate_up, W_down, W_shared_gate)


def _gather_body(routed_hbm, dest_hbm, out_hbm, dest_v, rows_v, sem):
    wid = lax.axis_index("s") * 2 + lax.axis_index("c")
    base = wid * TPW
    pltpu.sync_copy(dest_hbm.at[pl.ds(base, TPW)], dest_v)
    pltpu.async_copy(routed_hbm.at[dest_v], rows_v, sem).wait()
    pltpu.sync_copy(rows_v, out_hbm.at[pl.ds(base, TPW)])


def _sc_gather(routed_sorted, dest):
    mesh = plsc.VectorSubcoreMesh(core_axis_name="c", subcore_axis_name="s")
    fn = pl.kernel(
        _gather_body,
        out_type=jax.ShapeDtypeStruct((T, D), jnp.float32),
        mesh=mesh,
        scratch_types=[
            pltpu.VMEM((TPW,), jnp.int32),
            pltpu.VMEM((TPW, D), jnp.float32),
            pltpu.SemaphoreType.DMA,
        ],
    )
    return fn(routed_sorted, dest)


CB = 512


def _combine_body(r_ref, v_ref, sh_ref, o_ref):
    o_ref[...] = v_ref[...] * r_ref[...] + sh_ref[...]


def _run_combine(routed, val, shared):
    return pl.pallas_call(
        _combine_body,
        grid=(T // CB,),
        in_specs=[
            pl.BlockSpec((CB, D), lambda b: (b, 0)),
            pl.BlockSpec((CB, 1), lambda b: (b, 0)),
            pl.BlockSpec((CB, D), lambda b: (b, 0)),
        ],
        out_specs=pl.BlockSpec((CB, D), lambda b: (b, 0)),
        out_shape=jax.ShapeDtypeStruct((T, D), jnp.float32),
        compiler_params=pltpu.CompilerParams(
            dimension_semantics=("arbitrary",)),
    )(routed, val, shared)


def kernel(x, W_router, W1, W2, W_gate_up, W_down, W_shared_gate):
    Bb, Tt, Dd = x.shape
    x2 = x.reshape(Tt, Dd)
    val2, tile_e, ntl, dest2 = _run_router(x2, W_router)
    dest = dest2.reshape(Tt)
    sorted_x = _sc_scatter(x2, dest)
    routed_sorted, shared = _run_experts(
        tile_e.reshape(16), ntl.reshape(1), sorted_x, W1, W2,
        x2, W_gate_up, W_down, W_shared_gate)
    routed = _sc_gather(routed_sorted, dest)
    out = _run_combine(routed, val2, shared)
    return out.reshape(Bb, Tt, Dd)
